# baseline (speedup 1.0000x reference)
import functools

import jax
import jax.numpy as jnp
from jax import lax
from jax.experimental import pallas as pl
from jax.experimental.pallas import tpu as pltpu
from jax.experimental.pallas import tpu_sc as plsc

N = 1024
H = 128
O = 64
L = 2
HC = H // 16
NC, NS = 2, 16
NW = NC * NS
JPW = N // NW
CHUNK = 256
NCH = N // CHUNK


def _proj_body(x_ref, Wk_ref, bk_ref, Wq_ref, bq_ref, Wv_ref, bv_ref,
               Ws_ref, b_ref, ek_ref, eq_ref, v_ref, s_ref):
    f32 = jnp.float32
    x = x_ref[:]
    ek_ref[:] = jnp.minimum(jnp.exp(
        -(jnp.dot(x, Wk_ref[:], preferred_element_type=f32) + bk_ref[0:1, :])), 1e30)
    eq_ref[:] = jnp.minimum(jnp.exp(
        -(jnp.dot(x, Wq_ref[:], preferred_element_type=f32) + bq_ref[0:1, :])), 1e30)
    v_ref[:] = jnp.dot(x, Wv_ref[:], preferred_element_type=f32) + bv_ref[0:1, :]
    s_ref[:] = jnp.dot(x, Ws_ref[:], preferred_element_type=f32) + b_ref[0:1, :]


def _proj(x, Wk, bk, Wq, bq, Wv, bv, Ws, b):
    return pl.pallas_call(
        _proj_body,
        out_shape=[jax.ShapeDtypeStruct((N, H), jnp.float32)] * 4,
    )(x, Wk, bk, Wq, bq, Wv, bv, Ws, b)


def _post_body(agg_ref, s_ref, mW_ref, mb_ref, lng_ref, lnb_ref, x_ref):
    f32 = jnp.float32
    xa = agg_ref[:] + s_ref[:]
    h1 = jnp.dot(xa, mW_ref[:], preferred_element_type=f32) + mb_ref[0:1, :]
    mu = jnp.mean(h1, axis=-1, keepdims=True)
    var = jnp.mean((h1 - mu) ** 2, axis=-1, keepdims=True)
    hn = (h1 - mu) / jnp.sqrt(var + 1e-5) * lng_ref[0:1, :] + lnb_ref[0:1, :]
    x_ref[:] = jnp.maximum(hn, 0.0)


def _post(agg, s, mW, mb, lng, lnb):
    return pl.pallas_call(
        _post_body,
        out_shape=jax.ShapeDtypeStruct((N, H), jnp.float32),
    )(agg, s, mW, mb, lng, lnb)


def _heads_body(x_ref, linW_ref, linb_ref, m1W_ref, m1b_ref, m2W_ref,
                m2b_ref, mu_ref, lv_ref):
    f32 = jnp.float32
    x = x_ref[:]
    mu_ref[:] = jnp.dot(x, linW_ref[:], preferred_element_type=f32) + linb_ref[0:1, :]
    h = jnp.maximum(jnp.dot(x, m1W_ref[:], preferred_element_type=f32) + m1b_ref[0:1, :], 0.0)
    lv_ref[:] = jnp.dot(h, m2W_ref[:], preferred_element_type=f32) + m2b_ref[0:1, :]


def _heads(x, linW, linb, m1W, m1b, m2W, m2b):
    return pl.pallas_call(
        _heads_body,
        out_shape=[jax.ShapeDtypeStruct((N, O), jnp.float32)] * 2,
    )(x, linW, linb, m1W, m1b, m2W, m2b)


def _sc_agg_body(ek_hbm, eq_hbm, v_hbm, at_hbm, out_hbm, ekt, mt, eqc, vc, accv):
    wid = lax.axis_index("s") * NC + lax.axis_index("c")
    base = wid * JPW
    pltpu.sync_copy(ek_hbm.at[pl.ds(base, JPW)], ekt)
    pltpu.sync_copy(at_hbm.at[pl.ds(base, JPW)], mt)
    for ch in range(NCH):
        pltpu.sync_copy(eq_hbm.at[pl.ds(ch * CHUNK, CHUNK)], eqc)
        pltpu.sync_copy(v_hbm.at[pl.ds(ch * CHUNK, CHUNK)], vc)

        def jbody(j, _, ch=ch):
            ekv = tuple(ekt[j, pl.ds(hc * 16, 16)] for hc in range(HC))

            def igbody(ig, acc, j=j, ch=ch, ekv=ekv):
                m16 = mt[j, pl.ds(ch * CHUNK + ig * 16, 16)]
                out = list(acc)
                for li in range(16):
                    i = ig * 16 + li
                    msc = m16[li]
                    for hc in range(HC):
                        d = 1.0 + ekv[hc] * eqc[i, pl.ds(hc * 16, 16)]
                        out[hc] = out[hc] + (msc * vc[i, pl.ds(hc * 16, 16)]) / d
                return tuple(out)

            acc = lax.fori_loop(
                0, CHUNK // 16, igbody,
                tuple(jnp.zeros((16,), jnp.float32) for _ in range(HC)))
            for hc in range(HC):
                if ch == 0:
                    accv[j, pl.ds(hc * 16, 16)] = acc[hc]
                else:
                    plsc.addupdate(accv.at[j, pl.ds(hc * 16, 16)], acc[hc])
            return 0

        lax.fori_loop(0, JPW, jbody, 0)
    pltpu.sync_copy(accv, out_hbm.at[pl.ds(base, JPW)])


@functools.cache
def _make_sc_agg():
    mesh = plsc.VectorSubcoreMesh(core_axis_name="c", subcore_axis_name="s")
    return pl.kernel(
        _sc_agg_body, mesh=mesh,
        out_type=jax.ShapeDtypeStruct((N, H), jnp.float32),
        scratch_types=[
            pltpu.VMEM((JPW, H), jnp.float32),
            pltpu.VMEM((JPW, N), jnp.float32),
            pltpu.VMEM((CHUNK, H), jnp.float32),
            pltpu.VMEM((CHUNK, H), jnp.float32),
            pltpu.VMEM((JPW, H), jnp.float32),
        ],
    )


@jax.jit
def _decoder(x, AT, Wk, bk, Wq, bq, Wv, bv, Ws, b, mW, mb, lng, lnb,
             linW, linb, m1W, m1b, m2W, m2b):
    sc_agg = _make_sc_agg()
    for l in range(L):
        ek, eq, v, s = _proj(x, Wk[l], bk[l:l + 1], Wq[l], bq[l:l + 1],
                             Wv[l], bv[l:l + 1], Ws[l], b[l:l + 1])
        agg = sc_agg(ek, eq, v, AT)
        x = _post(agg, s, mW[l], mb[l:l + 1],
                  lng[l:l + 1], lnb[l:l + 1])
    return _heads(x, linW, linb.reshape(1, O), m1W, m1b.reshape(1, H),
                  m2W, m2b.reshape(1, O))


def kernel(node_feat, adj, Wk, bk, Wq, bq, Wv, bv, Ws, b, mW, mb, lng, lnb,
           linW, linb, m1W, m1b, m2W, m2b, grad_out=None):
    x = node_feat[0]
    AT = adj[0].T
    mu, lv = _decoder(x, AT, Wk, bk, Wq, bq, Wv, bv, Ws, b, mW, mb, lng, lnb,
                      linW, linb, m1W, m1b, m2W, m2b)
    return (mu[None], lv[None])


# speedup vs baseline: 2.4882x; 2.4882x over previous
import functools

import jax
import jax.numpy as jnp
from jax import lax
from jax.experimental import pallas as pl
from jax.experimental.pallas import tpu as pltpu
from jax.experimental.pallas import tpu_sc as plsc

N = 1024
H = 128
O = 64
L = 2
HC = H // 16
NC, NS = 2, 16
NW = NC * NS
NSC = 256
JPW = NSC // NW
CHUNK = 256
NCH = N // CHUNK
TI = 16
TJ = 128
NTC = N - NSC


def _proj_body(x_ref, Wk_ref, bk_ref, Wq_ref, bq_ref, Wv_ref, bv_ref,
               Ws_ref, b_ref, ek_ref, eq_ref, v_ref, s_ref):
    f32 = jnp.float32
    x = x_ref[:]
    ek_ref[:] = jnp.minimum(jnp.exp(
        -(jnp.dot(x, Wk_ref[:], preferred_element_type=f32) + bk_ref[0:1, :])), 1e30)
    eq_ref[:] = jnp.minimum(jnp.exp(
        -(jnp.dot(x, Wq_ref[:], preferred_element_type=f32) + bq_ref[0:1, :])), 1e30)
    v_ref[:] = jnp.dot(x, Wv_ref[:], preferred_element_type=f32) + bv_ref[0:1, :]
    s_ref[:] = jnp.dot(x, Ws_ref[:], preferred_element_type=f32) + b_ref[0:1, :]


def _proj(x, Wk, bk, Wq, bq, Wv, bv, Ws, b):
    return pl.pallas_call(
        _proj_body,
        out_shape=[jax.ShapeDtypeStruct((N, H), jnp.float32)] * 4,
    )(x, Wk, bk, Wq, bq, Wv, bv, Ws, b)


def _post_body(aggs_ref, aggt_ref, s_ref, mW_ref, mb_ref, lng_ref, lnb_ref, x_ref):
    f32 = jnp.float32
    xa = jnp.concatenate([aggs_ref[:], aggt_ref[:]], axis=0) + s_ref[:]
    h1 = jnp.dot(xa, mW_ref[:], preferred_element_type=f32) + mb_ref[0:1, :]
    mu = jnp.mean(h1, axis=-1, keepdims=True)
    var = jnp.mean((h1 - mu) ** 2, axis=-1, keepdims=True)
    hn = (h1 - mu) / jnp.sqrt(var + 1e-5) * lng_ref[0:1, :] + lnb_ref[0:1, :]
    x_ref[:] = jnp.maximum(hn, 0.0)


def _post(aggs, aggt, s, mW, mb, lng, lnb):
    return pl.pallas_call(
        _post_body,
        out_shape=jax.ShapeDtypeStruct((N, H), jnp.float32),
    )(aggs, aggt, s, mW, mb, lng, lnb)


def _heads_body(x_ref, linW_ref, linb_ref, m1W_ref, m1b_ref, m2W_ref,
                m2b_ref, mu_ref, lv_ref):
    f32 = jnp.float32
    x = x_ref[:]
    mu_ref[:] = jnp.dot(x, linW_ref[:], preferred_element_type=f32) + linb_ref[0:1, :]
    h = jnp.maximum(jnp.dot(x, m1W_ref[:], preferred_element_type=f32) + m1b_ref[0:1, :], 0.0)
    lv_ref[:] = jnp.dot(h, m2W_ref[:], preferred_element_type=f32) + m2b_ref[0:1, :]


def _heads(x, linW, linb, m1W, m1b, m2W, m2b):
    return pl.pallas_call(
        _heads_body,
        out_shape=[jax.ShapeDtypeStruct((N, O), jnp.float32)] * 2,
    )(x, linW, linb, m1W, m1b, m2W, m2b)


def _tc_agg_body(ek_ref, eq_ref, v_ref, A_ref, agg_ref):
    f32 = jnp.float32
    for jt in range(NTC // TJ):
        kt = ek_ref[NSC + jt * TJ:NSC + (jt + 1) * TJ, :]

        def ibody(it, acc, kt=kt, jt=jt):
            row = pl.multiple_of(it * TI, TI)
            qt = eq_ref[pl.ds(row, TI), :]
            vt = v_ref[pl.ds(row, TI), :]
            Mt = A_ref[pl.ds(row, TI), NSC + jt * TJ:NSC + (jt + 1) * TJ]
            d = 1.0 + kt[None, :, :] * qt[:, None, :]
            msg = (vt[:, None, :] / d) * Mt[:, :, None]
            return acc + jnp.sum(msg, axis=0)

        agg = lax.fori_loop(0, N // TI, ibody, jnp.zeros((TJ, H), f32))
        agg_ref[jt * TJ:(jt + 1) * TJ, :] = agg


def _tc_agg(ek, eq, v, A):
    return pl.pallas_call(
        _tc_agg_body,
        out_shape=jax.ShapeDtypeStruct((NTC, H), jnp.float32),
    )(ek, eq, v, A)


def _sc_agg_body(ek_hbm, eq_hbm, v_hbm, at_hbm, out_hbm, ekt, mt, eqc, vc, accv):
    wid = lax.axis_index("s") * NC + lax.axis_index("c")
    base = wid * JPW
    pltpu.sync_copy(ek_hbm.at[pl.ds(base, JPW)], ekt)
    pltpu.sync_copy(at_hbm.at[pl.ds(base, JPW)], mt)
    for ch in range(NCH):
        pltpu.sync_copy(eq_hbm.at[pl.ds(ch * CHUNK, CHUNK)], eqc)
        pltpu.sync_copy(v_hbm.at[pl.ds(ch * CHUNK, CHUNK)], vc)

        def jbody(j, _, ch=ch):
            ekv = tuple(ekt[j, pl.ds(hc * 16, 16)] for hc in range(HC))

            def igbody(ig, acc, j=j, ch=ch, ekv=ekv):
                m16 = mt[j, pl.ds(ch * CHUNK + ig * 16, 16)]
                out = list(acc)
                for li in range(16):
                    i = ig * 16 + li
                    msc = m16[li]
                    for hc in range(HC):
                        d = 1.0 + ekv[hc] * eqc[i, pl.ds(hc * 16, 16)]
                        out[hc] = out[hc] + (msc * vc[i, pl.ds(hc * 16, 16)]) / d
                return tuple(out)

            acc = lax.fori_loop(
                0, CHUNK // 16, igbody,
                tuple(jnp.zeros((16,), jnp.float32) for _ in range(HC)))
            for hc in range(HC):
                if ch == 0:
                    accv[j, pl.ds(hc * 16, 16)] = acc[hc]
                else:
                    plsc.addupdate(accv.at[j, pl.ds(hc * 16, 16)], acc[hc])
            return 0

        lax.fori_loop(0, JPW, jbody, 0)
    pltpu.sync_copy(accv, out_hbm.at[pl.ds(base, JPW)])


@functools.cache
def _make_sc_agg():
    mesh = plsc.VectorSubcoreMesh(core_axis_name="c", subcore_axis_name="s")
    return pl.kernel(
        _sc_agg_body, mesh=mesh,
        out_type=jax.ShapeDtypeStruct((NSC, H), jnp.float32),
        scratch_types=[
            pltpu.VMEM((JPW, H), jnp.float32),
            pltpu.VMEM((JPW, N), jnp.float32),
            pltpu.VMEM((CHUNK, H), jnp.float32),
            pltpu.VMEM((CHUNK, H), jnp.float32),
            pltpu.VMEM((JPW, H), jnp.float32),
        ],
    )


@jax.jit
def _decoder(x, A, AT, Wk, bk, Wq, bq, Wv, bv, Ws, b, mW, mb, lng, lnb,
             linW, linb, m1W, m1b, m2W, m2b):
    sc_agg = _make_sc_agg()
    for l in range(L):
        ek, eq, v, s = _proj(x, Wk[l], bk[l:l + 1], Wq[l], bq[l:l + 1],
                             Wv[l], bv[l:l + 1], Ws[l], b[l:l + 1])
        agg_sc = sc_agg(ek, eq, v, AT)
        agg_tc = _tc_agg(ek, eq, v, A)
        x = _post(agg_sc, agg_tc, s, mW[l], mb[l:l + 1],
                  lng[l:l + 1], lnb[l:l + 1])
    return _heads(x, linW, linb.reshape(1, O), m1W, m1b.reshape(1, H),
                  m2W, m2b.reshape(1, O))


def kernel(node_feat, adj, Wk, bk, Wq, bq, Wv, bv, Ws, b, mW, mb, lng, lnb,
           linW, linb, m1W, m1b, m2W, m2b, grad_out=None):
    x = node_feat[0]
    A = adj[0]
    AT = A.T
    mu, lv = _decoder(x, A, AT, Wk, bk, Wq, bq, Wv, bv, Ws, b, mW, mb, lng, lnb,
                      linW, linb, m1W, m1b, m2W, m2b)
    return (mu[None], lv[None])
